# baseline (speedup 1.0000x reference)
import functools

import jax
import jax.numpy as jnp
from jax import lax
from jax.experimental import pallas as pl
from jax.experimental.pallas import tpu as pltpu
from jax.experimental.pallas import tpu_sc as plsc

N_NODES = 10000
NPAD = 10240
NC, NS = 2, 16
NW = NC * NS
CH = 128
ROWS_PER_TILE = NPAD // NS

_SC_PARAMS = pltpu.CompilerParams(use_tc_tiling_on_sc=False)
_SC_COUNT_PARAMS = pltpu.CompilerParams(use_tc_tiling_on_sc=False,
                                        needs_layout_passes=False)
L = 16


@functools.lru_cache(maxsize=None)
def _counts_kernel(n_ch):
    mesh = plsc.VectorSubcoreMesh(core_axis_name="c", subcore_axis_name="s")

    @functools.partial(
        pl.kernel,
        mesh=mesh,
        compiler_params=_SC_COUNT_PARAMS,
        out_type=jax.ShapeDtypeStruct((2, NC, NS, NPAD), jnp.float32),
        scratch_types=[
            pltpu.VMEM((n_ch, CH), jnp.int32),
            pltpu.VMEM((NPAD,), jnp.float32),
        ],
    )
    def k(dst_hbm, cnt_hbm, dstv, cnt_v):
        cid = lax.axis_index("c")
        sid = lax.axis_index("s")
        wid = sid * NC + cid
        zeros16 = jnp.zeros((L,), jnp.float32)
        ones16 = jnp.ones((L,), jnp.float32)

        for layer in range(2):
            pltpu.sync_copy(dst_hbm.at[layer, wid], dstv)

            @pl.loop(0, NPAD // L)
            def _(i):
                cnt_v[pl.ds(i * L, L)] = zeros16

            @pl.loop(0, n_ch)
            def _(ci):
                for j in range(CH // L):
                    d = dstv[ci, pl.ds(j * L, L)]
                    plsc.addupdate_scatter(cnt_v, [d], ones16)

            pltpu.sync_copy(cnt_v, cnt_hbm.at[layer, cid, sid])

    return k


def _prep_edges(edge_index):
    src, dst = edge_index[0], edge_index[1]
    e = src.shape[0]
    e_pad = -(-e // (NW * CH * 2)) * (NW * CH * 2)
    pad = e_pad - e
    pad_i = N_NODES + (jnp.arange(pad, dtype=jnp.int32) % (NPAD - N_NODES))
    src = jnp.concatenate([src, pad_i])
    dst = jnp.concatenate([dst, pad_i])
    n_ch = e_pad // (NW * CH)
    return src.reshape(NW, n_ch, CH), dst.reshape(NW, n_ch, CH), n_ch


@functools.lru_cache(maxsize=None)
def _segsum_kernel(n_ch, width):
    mesh = plsc.VectorSubcoreMesh(core_axis_name="c", subcore_axis_name="s")

    @functools.partial(
        pl.kernel,
        mesh=mesh,
        compiler_params=_SC_PARAMS,
        out_type=jax.ShapeDtypeStruct((NC, NPAD, width), jnp.float32),
        scratch_types=[
            pltpu.VMEM((n_ch, CH), jnp.int32),
            pltpu.VMEM((1, CH), jnp.int32),
            pltpu.VMEM((1, CH), jnp.int32),
            pltpu.VMEM((CH, width), jnp.float32),
            pltpu.VMEM((CH, width), jnp.float32),
            pltpu.VMEM_SHARED((NPAD, width), jnp.float32),
            pltpu.SemaphoreType.DMA,
            pltpu.SemaphoreType.DMA,
            pltpu.SemaphoreType.DMA,
            pltpu.SemaphoreType.DMA,
            pltpu.SemaphoreType.DMA,
            pltpu.SemaphoreType.DMA,
        ],
    )
    def k(table_hbm, src_hbm, dst_hbm, z_hbm, acc_hbm, srcv, d_a, d_b,
          rows_a, rows_b, acc, sem_ga, sem_gb, sem_da, sem_db, sem_sa, sem_sb):
        cid = lax.axis_index("c")
        sid = lax.axis_index("s")
        wid = sid * NC + cid
        pltpu.sync_copy(src_hbm.at[wid], srcv)
        row0 = sid * ROWS_PER_TILE
        pltpu.sync_copy(z_hbm, acc.at[pl.ds(row0, ROWS_PER_TILE)])
        plsc.subcore_barrier()

        pltpu.async_copy(dst_hbm.at[wid, 0], d_a, sem_da)
        pltpu.async_copy(dst_hbm.at[wid, 1], d_b, sem_db)
        pltpu.async_copy(table_hbm.at[srcv.at[0]], rows_a, sem_ga)
        pltpu.async_copy(table_hbm.at[srcv.at[1]], rows_b, sem_gb)

        @pl.loop(0, n_ch, step=2)
        def _(ci):
            pltpu.make_async_copy(table_hbm.at[srcv.at[ci]], rows_a, sem_ga).wait()
            pltpu.make_async_copy(dst_hbm.at[wid, ci], d_a, sem_da).wait()
            pltpu.async_copy(rows_a, acc.at[d_a.at[0]], sem_sa, add=True)

            pltpu.make_async_copy(table_hbm.at[srcv.at[ci + 1]], rows_b, sem_gb).wait()
            pltpu.make_async_copy(dst_hbm.at[wid, ci + 1], d_b, sem_db).wait()
            pltpu.async_copy(rows_b, acc.at[d_b.at[0]], sem_sb, add=True)

            pltpu.make_async_copy(rows_a, acc.at[d_a.at[0]], sem_sa).wait()

            @pl.when(ci + 2 < n_ch)
            def _():
                pltpu.async_copy(table_hbm.at[srcv.at[ci + 2]], rows_a, sem_ga)
                pltpu.async_copy(dst_hbm.at[wid, ci + 2], d_a, sem_da)

            pltpu.make_async_copy(rows_b, acc.at[d_b.at[0]], sem_sb).wait()

            @pl.when(ci + 3 < n_ch)
            def _():
                pltpu.async_copy(table_hbm.at[srcv.at[ci + 3]], rows_b, sem_gb)
                pltpu.async_copy(dst_hbm.at[wid, ci + 3], d_b, sem_db)

        plsc.subcore_barrier()
        pltpu.sync_copy(acc.at[pl.ds(row0, ROWS_PER_TILE)],
                        acc_hbm.at[cid, pl.ds(row0, ROWS_PER_TILE)])

    return k


def _segsum(table, src3, dst3, n_ch, width):
    zeros = jnp.zeros((ROWS_PER_TILE, width), jnp.float32)
    dst4 = dst3.reshape(NW, n_ch, 1, CH)
    return _segsum_kernel(n_ch, width)(table, src3, dst4, zeros)


def _tc_table1(x, w):
    def body(x_ref, w_ref, o_ref):
        y = jnp.dot(x_ref[...], w_ref[...], preferred_element_type=jnp.float32)
        o_ref[...] = jnp.pad(y, ((0, NPAD - N_NODES), (0, 0)))

    return pl.pallas_call(
        body, out_shape=jax.ShapeDtypeStruct((NPAD, 128), jnp.float32))(x, w)


def _tc_mid(p1, c1, x, w1r, b1, w2l, w2r):
    def body(p_ref, c_ref, x_ref, wr_ref, b_ref, wl2_ref, wr2_ref, t2_ref, hr_ref):
        agg = p_ref[0, :N_NODES] + p_ref[1, :N_NODES]
        cnt = jnp.sum(c_ref[...], axis=0)[:N_NODES]
        cntc = jnp.maximum(cnt, 1.0).reshape(N_NODES, 1)
        h = agg / cntc + b_ref[...] + jnp.dot(
            x_ref[...], wr_ref[...], preferred_element_type=jnp.float32)
        h = jnp.maximum(h, 0.0)
        y2 = jnp.dot(h, wl2_ref[...], preferred_element_type=jnp.float32)
        t2_ref[...] = jnp.pad(y2, ((0, NPAD - N_NODES), (0, 0)))
        hr_ref[...] = jnp.dot(h, wr2_ref[...], preferred_element_type=jnp.float32)

    return pl.pallas_call(
        body,
        out_shape=[jax.ShapeDtypeStruct((NPAD, 64), jnp.float32),
                   jax.ShapeDtypeStruct((N_NODES, 64), jnp.float32)],
    )(p1, c1.reshape(NC * NS, NPAD), x, w1r, b1.reshape(1, -1), w2l, w2r)


def _tc_out(p2, c2, hr, b2):
    def body(p_ref, c_ref, hr_ref, b_ref, o_ref):
        agg = p_ref[0, :N_NODES, :64] + p_ref[1, :N_NODES, :64]
        cnt = jnp.sum(c_ref[...], axis=0)[:N_NODES]
        cntc = jnp.maximum(cnt, 1.0).reshape(N_NODES, 1)
        o_ref[...] = agg / cntc + b_ref[...] + hr_ref[...]

    return pl.pallas_call(
        body, out_shape=jax.ShapeDtypeStruct((N_NODES, 64), jnp.float32))(
            p2, c2.reshape(NC * NS, NPAD), hr, b2.reshape(1, -1))


def kernel(x, edge_index1, edge_index2, W1_l, W1_r, b1, W2_l, W2_r, b2):
    src1, dst1, n1 = _prep_edges(edge_index1)
    src2, dst2, n2 = _prep_edges(edge_index2)
    cnt_all = _counts_kernel(n1)(jnp.stack([dst1, dst2]))
    t1 = _tc_table1(x, W1_l)
    p1 = _segsum(t1, src1, dst1, n1, 128)
    t2, hr = _tc_mid(p1, cnt_all[0], x, W1_r, b1, W2_l, W2_r)
    p2 = _segsum(t2, src2, dst2, n2, 64)
    return _tc_out(p2, cnt_all[1], hr, b2)


# speedup vs baseline: 1.1861x; 1.1861x over previous
import functools

import jax
import jax.numpy as jnp
from jax import lax
from jax.experimental import pallas as pl
from jax.experimental.pallas import tpu as pltpu
from jax.experimental.pallas import tpu_sc as plsc

N_NODES = 10000
NPAD = 10240
NC, NS = 2, 16
NW = NC * NS
CH = 128
ROWS_PER_TILE = NPAD // NS
L = 16

_SC_PARAMS = pltpu.CompilerParams(use_tc_tiling_on_sc=False,
                                  needs_layout_passes=False)


def _prep_edges(edge_index):
    src, dst = edge_index[0], edge_index[1]
    e = src.shape[0]
    e_pad = -(-e // (NW * CH * 2)) * (NW * CH * 2)
    pad = e_pad - e
    pad_i = N_NODES + (jnp.arange(pad, dtype=jnp.int32) % (NPAD - N_NODES))
    src = jnp.concatenate([src, pad_i])
    dst = jnp.concatenate([dst, pad_i])
    n_ch = e_pad // (NW * CH)
    return (src.reshape(NW, n_ch, 1, CH), dst.reshape(NW, n_ch, 1, CH), n_ch)


@functools.lru_cache(maxsize=None)
def _segsum_kernel(n_ch, width):
    mesh = plsc.VectorSubcoreMesh(core_axis_name="c", subcore_axis_name="s")

    @functools.partial(
        pl.kernel,
        mesh=mesh,
        compiler_params=_SC_PARAMS,
        out_type=[jax.ShapeDtypeStruct((NC, NPAD, width), jnp.float32),
                  jax.ShapeDtypeStruct((NC, NS, NPAD), jnp.float32)],
        scratch_types=[
            pltpu.VMEM((1, CH), jnp.int32),
            pltpu.VMEM((1, CH), jnp.int32),
            pltpu.VMEM((1, CH), jnp.int32),
            pltpu.VMEM((1, CH), jnp.int32),
            pltpu.VMEM((CH, width), jnp.float32),
            pltpu.VMEM((CH, width), jnp.float32),
            pltpu.VMEM((NPAD,), jnp.float32),
            pltpu.VMEM_SHARED((NPAD, width), jnp.float32),
            pltpu.SemaphoreType.DMA,
            pltpu.SemaphoreType.DMA,
            pltpu.SemaphoreType.DMA,
            pltpu.SemaphoreType.DMA,
            pltpu.SemaphoreType.DMA,
            pltpu.SemaphoreType.DMA,
        ],
    )
    def k(table_hbm, src_hbm, dst_hbm, z_hbm, acc_hbm, cnt_hbm,
          s_a, s_b, d_a, d_b, rows_a, rows_b, cnt_v, acc,
          sem_ia, sem_ib, sem_da, sem_db, sem_ga, sem_gb):
        cid = lax.axis_index("c")
        sid = lax.axis_index("s")
        wid = sid * NC + cid
        row0 = sid * ROWS_PER_TILE
        pltpu.async_copy(src_hbm.at[wid, 0], s_a, sem_ia)
        pltpu.async_copy(src_hbm.at[wid, 1], s_b, sem_ib)
        pltpu.async_copy(dst_hbm.at[wid, 0], d_a, sem_da)
        pltpu.async_copy(dst_hbm.at[wid, 1], d_b, sem_db)
        pltpu.sync_copy(z_hbm, acc.at[pl.ds(row0, ROWS_PER_TILE)])

        zeros16 = jnp.zeros((L,), jnp.float32)
        ones16 = jnp.ones((L,), jnp.float32)

        @pl.loop(0, NPAD // L)
        def _(i):
            cnt_v[pl.ds(i * L, L)] = zeros16

        plsc.subcore_barrier()
        pltpu.make_async_copy(src_hbm.at[wid, 0], s_a, sem_ia).wait()
        pltpu.async_copy(table_hbm.at[s_a.at[0]], rows_a, sem_ga)
        pltpu.make_async_copy(src_hbm.at[wid, 1], s_b, sem_ib).wait()
        pltpu.async_copy(table_hbm.at[s_b.at[0]], rows_b, sem_gb)

        @pl.loop(0, n_ch, step=2)
        def _(ci):
            pltpu.make_async_copy(table_hbm.at[s_a.at[0]], rows_a, sem_ga).wait()

            @pl.when(ci + 2 < n_ch)
            def _():
                pltpu.async_copy(src_hbm.at[wid, ci + 2], s_a, sem_ia)

            pltpu.make_async_copy(dst_hbm.at[wid, ci], d_a, sem_da).wait()
            pltpu.sync_copy(rows_a, acc.at[d_a.at[0]], add=True)
            for j in range(CH // L):
                plsc.addupdate_scatter(cnt_v, [d_a[0, pl.ds(j * L, L)]], ones16)

            @pl.when(ci + 2 < n_ch)
            def _():
                pltpu.make_async_copy(src_hbm.at[wid, ci + 2], s_a, sem_ia).wait()
                pltpu.async_copy(table_hbm.at[s_a.at[0]], rows_a, sem_ga)
                pltpu.async_copy(dst_hbm.at[wid, ci + 2], d_a, sem_da)

            pltpu.make_async_copy(table_hbm.at[s_b.at[0]], rows_b, sem_gb).wait()

            @pl.when(ci + 3 < n_ch)
            def _():
                pltpu.async_copy(src_hbm.at[wid, ci + 3], s_b, sem_ib)

            pltpu.make_async_copy(dst_hbm.at[wid, ci + 1], d_b, sem_db).wait()
            pltpu.sync_copy(rows_b, acc.at[d_b.at[0]], add=True)
            for j in range(CH // L):
                plsc.addupdate_scatter(cnt_v, [d_b[0, pl.ds(j * L, L)]], ones16)

            @pl.when(ci + 3 < n_ch)
            def _():
                pltpu.make_async_copy(src_hbm.at[wid, ci + 3], s_b, sem_ib).wait()
                pltpu.async_copy(table_hbm.at[s_b.at[0]], rows_b, sem_gb)
                pltpu.async_copy(dst_hbm.at[wid, ci + 3], d_b, sem_db)

        pltpu.sync_copy(cnt_v, cnt_hbm.at[cid, sid])
        plsc.subcore_barrier()
        pltpu.sync_copy(acc.at[pl.ds(row0, ROWS_PER_TILE)],
                        acc_hbm.at[cid, pl.ds(row0, ROWS_PER_TILE)])

    return k


def _segsum(table, src4, dst4, n_ch, width):
    zeros = jnp.zeros((ROWS_PER_TILE, width), jnp.float32)
    return _segsum_kernel(n_ch, width)(table, src4, dst4, zeros)


def _tc_table1(x, w):
    def body(x_ref, w_ref, o_ref):
        y = jnp.dot(x_ref[...], w_ref[...], preferred_element_type=jnp.float32)
        o_ref[...] = jnp.pad(y, ((0, NPAD - N_NODES), (0, 0)))

    return pl.pallas_call(
        body, out_shape=jax.ShapeDtypeStruct((NPAD, 128), jnp.float32))(x, w)


def _tc_mid(p1, c1, x, w1r, b1, w2l, w2r):
    def body(p_ref, c_ref, x_ref, wr_ref, b_ref, wl2_ref, wr2_ref, t2_ref, hr_ref):
        agg = p_ref[0, :N_NODES] + p_ref[1, :N_NODES]
        cnt = jnp.sum(c_ref[...], axis=0)[:N_NODES]
        cntc = jnp.maximum(cnt, 1.0).reshape(N_NODES, 1)
        h = agg / cntc + b_ref[...] + jnp.dot(
            x_ref[...], wr_ref[...], preferred_element_type=jnp.float32)
        h = jnp.maximum(h, 0.0)
        y2 = jnp.dot(h, wl2_ref[...], preferred_element_type=jnp.float32)
        t2_ref[...] = jnp.pad(y2, ((0, NPAD - N_NODES), (0, 0)))
        hr_ref[...] = jnp.dot(h, wr2_ref[...], preferred_element_type=jnp.float32)

    return pl.pallas_call(
        body,
        out_shape=[jax.ShapeDtypeStruct((NPAD, 64), jnp.float32),
                   jax.ShapeDtypeStruct((N_NODES, 64), jnp.float32)],
    )(p1, c1.reshape(NC * NS, NPAD), x, w1r, b1.reshape(1, -1), w2l, w2r)


def _tc_out(p2, c2, hr, b2):
    def body(p_ref, c_ref, hr_ref, b_ref, o_ref):
        agg = p_ref[0, :N_NODES] + p_ref[1, :N_NODES]
        cnt = jnp.sum(c_ref[...], axis=0)[:N_NODES]
        cntc = jnp.maximum(cnt, 1.0).reshape(N_NODES, 1)
        o_ref[...] = agg / cntc + b_ref[...] + hr_ref[...]

    return pl.pallas_call(
        body, out_shape=jax.ShapeDtypeStruct((N_NODES, 64), jnp.float32))(
            p2, c2.reshape(NC * NS, NPAD), hr, b2.reshape(1, -1))


def kernel(x, edge_index1, edge_index2, W1_l, W1_r, b1, W2_l, W2_r, b2):
    src1, dst1, n1 = _prep_edges(edge_index1)
    src2, dst2, n2 = _prep_edges(edge_index2)
    t1 = _tc_table1(x, W1_l)
    p1, c1 = _segsum(t1, src1, dst1, n1, 128)
    t2, hr = _tc_mid(p1, c1, x, W1_r, b1, W2_l, W2_r)
    p2, c2 = _segsum(t2, src2, dst2, n2, 64)
    return _tc_out(p2, c2, hr, b2)


# speedup vs baseline: 1.2036x; 1.0148x over previous
import functools

import jax
import jax.numpy as jnp
from jax import lax
from jax.experimental import pallas as pl
from jax.experimental.pallas import tpu as pltpu
from jax.experimental.pallas import tpu_sc as plsc

N_NODES = 10000
NPAD = 10240
NC, NS = 2, 16
NW = NC * NS
CH = 128
ROWS_PER_TILE = NPAD // NS

_SC_PARAMS = pltpu.CompilerParams(use_tc_tiling_on_sc=False)
_SC_COUNT_PARAMS = pltpu.CompilerParams(use_tc_tiling_on_sc=False,
                                        needs_layout_passes=False)
L = 16


@functools.lru_cache(maxsize=None)
def _counts_kernel(n_ch):
    mesh = plsc.VectorSubcoreMesh(core_axis_name="c", subcore_axis_name="s")

    @functools.partial(
        pl.kernel,
        mesh=mesh,
        compiler_params=_SC_COUNT_PARAMS,
        out_type=jax.ShapeDtypeStruct((2, NC, NS, NPAD), jnp.float32),
        scratch_types=[
            pltpu.VMEM((n_ch, CH), jnp.int32),
            pltpu.VMEM((NPAD,), jnp.float32),
        ],
    )
    def k(dst_hbm, cnt_hbm, dstv, cnt_v):
        cid = lax.axis_index("c")
        sid = lax.axis_index("s")
        wid = sid * NC + cid
        zeros16 = jnp.zeros((L,), jnp.float32)
        ones16 = jnp.ones((L,), jnp.float32)

        for layer in range(2):
            pltpu.sync_copy(dst_hbm.at[layer, wid], dstv)

            @pl.loop(0, NPAD // L)
            def _(i):
                cnt_v[pl.ds(i * L, L)] = zeros16

            @pl.loop(0, n_ch)
            def _(ci):
                for j in range(CH // L):
                    d = dstv[ci, pl.ds(j * L, L)]
                    plsc.addupdate_scatter(cnt_v, [d], ones16)

            pltpu.sync_copy(cnt_v, cnt_hbm.at[layer, cid, sid])

    return k


def _prep_edges(edge_index):
    src, dst = edge_index[0], edge_index[1]
    e = src.shape[0]
    e_pad = -(-e // (NW * CH * 2)) * (NW * CH * 2)
    pad = e_pad - e
    pad_i = N_NODES + (jnp.arange(pad, dtype=jnp.int32) % (NPAD - N_NODES))
    src = jnp.concatenate([src, pad_i])
    dst = jnp.concatenate([dst, pad_i])
    n_ch = e_pad // (NW * CH)
    return src.reshape(NW, n_ch, CH), dst.reshape(NW, n_ch, CH), n_ch


@functools.lru_cache(maxsize=None)
def _segsum_kernel(n_ch, width):
    mesh = plsc.VectorSubcoreMesh(core_axis_name="c", subcore_axis_name="s")

    @functools.partial(
        pl.kernel,
        mesh=mesh,
        compiler_params=_SC_PARAMS,
        out_type=jax.ShapeDtypeStruct((NC, NPAD, width), jnp.float32),
        scratch_types=[
            pltpu.VMEM((n_ch, CH), jnp.int32),
            pltpu.VMEM((1, CH), jnp.int32),
            pltpu.VMEM((1, CH), jnp.int32),
            pltpu.VMEM((CH, width), jnp.float32),
            pltpu.VMEM((CH, width), jnp.float32),
            pltpu.VMEM_SHARED((NPAD, width), jnp.float32),
            pltpu.SemaphoreType.DMA,
            pltpu.SemaphoreType.DMA,
            pltpu.SemaphoreType.DMA,
            pltpu.SemaphoreType.DMA,
        ],
    )
    def k(table_hbm, src_hbm, dst_hbm, z_hbm, acc_hbm, srcv, d_a, d_b,
          rows_a, rows_b, acc, sem_ga, sem_gb, sem_da, sem_db):
        cid = lax.axis_index("c")
        sid = lax.axis_index("s")
        wid = sid * NC + cid
        pltpu.sync_copy(src_hbm.at[wid], srcv)
        row0 = sid * ROWS_PER_TILE
        pltpu.sync_copy(z_hbm, acc.at[pl.ds(row0, ROWS_PER_TILE)])
        plsc.subcore_barrier()

        pltpu.async_copy(dst_hbm.at[wid, 0], d_a, sem_da)
        pltpu.async_copy(dst_hbm.at[wid, 1], d_b, sem_db)
        pltpu.async_copy(table_hbm.at[srcv.at[0]], rows_a, sem_ga)
        pltpu.async_copy(table_hbm.at[srcv.at[1]], rows_b, sem_gb)

        @pl.loop(0, n_ch, step=2)
        def _(ci):
            pltpu.make_async_copy(table_hbm.at[srcv.at[ci]], rows_a, sem_ga).wait()
            pltpu.make_async_copy(dst_hbm.at[wid, ci], d_a, sem_da).wait()
            pltpu.sync_copy(rows_a, acc.at[d_a.at[0]], add=True)

            @pl.when(ci + 2 < n_ch)
            def _():
                pltpu.async_copy(table_hbm.at[srcv.at[ci + 2]], rows_a, sem_ga)
                pltpu.async_copy(dst_hbm.at[wid, ci + 2], d_a, sem_da)

            pltpu.make_async_copy(table_hbm.at[srcv.at[ci + 1]], rows_b, sem_gb).wait()
            pltpu.make_async_copy(dst_hbm.at[wid, ci + 1], d_b, sem_db).wait()
            pltpu.sync_copy(rows_b, acc.at[d_b.at[0]], add=True)

            @pl.when(ci + 3 < n_ch)
            def _():
                pltpu.async_copy(table_hbm.at[srcv.at[ci + 3]], rows_b, sem_gb)
                pltpu.async_copy(dst_hbm.at[wid, ci + 3], d_b, sem_db)

        plsc.subcore_barrier()
        pltpu.sync_copy(acc.at[pl.ds(row0, ROWS_PER_TILE)],
                        acc_hbm.at[cid, pl.ds(row0, ROWS_PER_TILE)])

    return k


def _segsum(table, src3, dst3, n_ch, width):
    zeros = jnp.zeros((ROWS_PER_TILE, width), jnp.float32)
    dst4 = dst3.reshape(NW, n_ch, 1, CH)
    return _segsum_kernel(n_ch, width)(table, src3, dst4, zeros)


def _tc_table1(x, w):
    def body(x_ref, w_ref, o_ref):
        y = jnp.dot(x_ref[...], w_ref[...], preferred_element_type=jnp.float32)
        o_ref[...] = jnp.pad(y, ((0, NPAD - N_NODES), (0, 0)))

    return pl.pallas_call(
        body, out_shape=jax.ShapeDtypeStruct((NPAD, 128), jnp.float32))(x, w)


def _tc_mid(p1, c1, x, w1r, b1, w2l, w2r):
    def body(p_ref, c_ref, x_ref, wr_ref, b_ref, wl2_ref, wr2_ref, t2_ref, hr_ref):
        agg = p_ref[0, :N_NODES] + p_ref[1, :N_NODES]
        cnt = jnp.sum(c_ref[...], axis=0)[:N_NODES]
        cntc = jnp.maximum(cnt, 1.0).reshape(N_NODES, 1)
        h = agg / cntc + b_ref[...] + jnp.dot(
            x_ref[...], wr_ref[...], preferred_element_type=jnp.float32)
        h = jnp.maximum(h, 0.0)
        y2 = jnp.dot(h, wl2_ref[...], preferred_element_type=jnp.float32)
        t2_ref[...] = jnp.pad(y2, ((0, NPAD - N_NODES), (0, 0)))
        hr_ref[...] = jnp.dot(h, wr2_ref[...], preferred_element_type=jnp.float32)

    return pl.pallas_call(
        body,
        out_shape=[jax.ShapeDtypeStruct((NPAD, 64), jnp.float32),
                   jax.ShapeDtypeStruct((N_NODES, 64), jnp.float32)],
    )(p1, c1.reshape(NC * NS, NPAD), x, w1r, b1.reshape(1, -1), w2l, w2r)


def _tc_out(p2, c2, hr, b2):
    def body(p_ref, c_ref, hr_ref, b_ref, o_ref):
        agg = p_ref[0, :N_NODES, :64] + p_ref[1, :N_NODES, :64]
        cnt = jnp.sum(c_ref[...], axis=0)[:N_NODES]
        cntc = jnp.maximum(cnt, 1.0).reshape(N_NODES, 1)
        o_ref[...] = agg / cntc + b_ref[...] + hr_ref[...]

    return pl.pallas_call(
        body, out_shape=jax.ShapeDtypeStruct((N_NODES, 64), jnp.float32))(
            p2, c2.reshape(NC * NS, NPAD), hr, b2.reshape(1, -1))


def kernel(x, edge_index1, edge_index2, W1_l, W1_r, b1, W2_l, W2_r, b2):
    src1, dst1, n1 = _prep_edges(edge_index1)
    src2, dst2, n2 = _prep_edges(edge_index2)
    cnt_all = _counts_kernel(n1)(jnp.stack([dst1, dst2]))
    t1 = _tc_table1(x, W1_l)
    p1 = _segsum(t1, src1, dst1, n1, 128)
    t2, hr = _tc_mid(p1, cnt_all[0], x, W1_r, b1, W2_l, W2_r)
    p2 = _segsum(t2, src2, dst2, n2, 64)
    return _tc_out(p2, cnt_all[1], hr, b2)
